# Initial kernel scaffold; baseline (speedup 1.0000x reference)
import jax
import jax.numpy as jnp
from jax.experimental import pallas as pl


def kernel(edge_attr, edge_index, init_W1, init_b1, init_W2, init_b2, init_g, init_bt, node_W1, node_b1, node_W2, node_b2, node_g, node_bt, edge_W1, edge_b1, edge_W2, edge_b2, edge_g, edge_bt):
    raise NotImplementedError("write your pallas kernel here")


# speedup vs baseline: 2.1169x; 2.1169x over previous
import functools

import jax
import jax.numpy as jnp
from jax import lax
from jax.experimental import pallas as pl
from jax.experimental.pallas import tpu as pltpu
from jax.experimental.pallas import tpu_sc as plsc

E = 320000
N = 10000
D = 128
BT = 2560
NC, NS = 2, 16
L = 16
CW = 16
_EPS = 1e-5

OWN = 5120
TR = 5248
RPT = TR // NS
WPT = OWN // NS
CHS = 128
GCH = E // CHS

CH = 80
EPW = E // (NC * NS)
NCH = EPW // CH


def _elu(x):
    return jnp.where(x > 0, x, jnp.exp(jnp.minimum(x, 0.0)) - 1.0)


def _s1_body(ea, w1, b1, w2, b2, h2o, st):
    i = pl.program_id(0)
    h = _elu(jnp.dot(ea[...], w1[...], preferred_element_type=jnp.float32) + b1[...])
    h = _elu(jnp.dot(h, w2[...], preferred_element_type=jnp.float32) + b2[...])
    h2o[...] = h

    @pl.when(i == 0)
    def _():
        st[...] = jnp.zeros_like(st)

    st[0:1, :] += jnp.sum(h, axis=0, keepdims=True)
    st[1:2, :] += jnp.sum(h * h, axis=0, keepdims=True)


_stage1 = pl.pallas_call(
    _s1_body,
    grid=(E // BT,),
    in_specs=[pl.BlockSpec((BT, D), lambda i: (i, 0)),
              pl.BlockSpec((D, D), lambda i: (0, 0)),
              pl.BlockSpec((1, D), lambda i: (0, 0)),
              pl.BlockSpec((D, D), lambda i: (0, 0)),
              pl.BlockSpec((1, D), lambda i: (0, 0))],
    out_specs=[pl.BlockSpec((BT, D), lambda i: (i, 0)),
               pl.BlockSpec((8, D), lambda i: (0, 0))],
    out_shape=[jax.ShapeDtypeStruct((E, D), jnp.float32),
               jax.ShapeDtypeStruct((8, D), jnp.float32)],
)


def _remap_chunk(idx_v, nbase):
    for j in range(CHS // L):
        v = idx_v[pl.ds(j * L, L)]
        loc = v - nbase
        ok = (loc >= 0) & (loc < OWN)
        idx_v[pl.ds(j * L, L)] = jnp.where(ok, loc, OWN)


def _n_chunks(sid):
    extra = GCH - (GCH // NS) * NS
    return GCH // NS + jnp.where(sid < extra, 1, 0)


def _seg_scatter_body(h2_hbm, dst_hbm, zrow_hbm,
                      seg_out, rows_v, idx_v, stage_v, table_s):
    cid = lax.axis_index("c")
    sid = lax.axis_index("s")
    nbase = cid * OWN

    pltpu.sync_copy(zrow_hbm, stage_v)
    pltpu.sync_copy(stage_v, table_s.at[pl.ds(sid * RPT, RPT), :])
    plsc.subcore_barrier()

    def body(i, carry):
        off = (sid + i * NS) * CHS
        pltpu.sync_copy(dst_hbm.at[pl.ds(off, CHS)], idx_v)
        pltpu.sync_copy(h2_hbm.at[pl.ds(off, CHS), :], rows_v)
        for j in range(CHS // L):
            v = idx_v[pl.ds(j * L, L)]
            loc = v - nbase
            ok = (loc >= 0) & (loc < OWN)
            loc = jnp.where(ok, loc, OWN)
            pltpu.sync_copy(rows_v.at[pl.ds(j * L, L), :],
                            table_s.at[loc], add=True)
        return carry

    lax.fori_loop(0, _n_chunks(sid), body, 0)
    plsc.subcore_barrier()

    pltpu.sync_copy(table_s.at[pl.ds(sid * WPT, WPT), :],
                    stage_v.at[pl.ds(0, WPT), :])
    pltpu.sync_copy(stage_v.at[pl.ds(0, WPT), :],
                    seg_out.at[cid, pl.ds(sid * WPT, WPT), :])


@functools.cache
def _mk_seg_scatter():
    mesh = plsc.VectorSubcoreMesh(core_axis_name="c", subcore_axis_name="s",
                                  num_cores=NC, num_subcores=NS)
    return pl.kernel(
        _seg_scatter_body,
        out_type=jax.ShapeDtypeStruct((NC, OWN, D), jnp.float32),
        mesh=mesh,
        scratch_types=[pltpu.VMEM((CHS, D), jnp.float32),
                       pltpu.VMEM((CHS,), jnp.int32),
                       pltpu.VMEM((RPT, D), jnp.float32),
                       pltpu.VMEM_SHARED((TR, D), jnp.float32)])


def _cnt_scatter_body(dst_hbm, zcnt_hbm, ones_hbm,
                      cnt_out, idx_v, ones_v, stagec_v, cnt_s):
    cid = lax.axis_index("c")
    sid = lax.axis_index("s")
    nbase = cid * OWN

    pltpu.sync_copy(zcnt_hbm, stagec_v)
    pltpu.sync_copy(stagec_v, cnt_s.at[pl.ds(sid * RPT, RPT), :])
    pltpu.sync_copy(ones_hbm, ones_v)
    plsc.subcore_barrier()

    def body(i, carry):
        off = (sid + i * NS) * CHS
        pltpu.sync_copy(dst_hbm.at[pl.ds(off, CHS)], idx_v)
        _remap_chunk(idx_v, nbase)
        pltpu.sync_copy(ones_v, cnt_s.at[idx_v], add=True)
        return carry

    lax.fori_loop(0, _n_chunks(sid), body, 0)
    plsc.subcore_barrier()

    pltpu.sync_copy(cnt_s.at[pl.ds(sid * WPT, WPT), :],
                    stagec_v.at[pl.ds(0, WPT), :])
    pltpu.sync_copy(stagec_v.at[pl.ds(0, WPT), :],
                    cnt_out.at[cid, pl.ds(sid * WPT, WPT), :])


@functools.cache
def _mk_cnt_scatter():
    mesh = plsc.VectorSubcoreMesh(core_axis_name="c", subcore_axis_name="s",
                                  num_cores=NC, num_subcores=NS)
    return pl.kernel(
        _cnt_scatter_body,
        out_type=jax.ShapeDtypeStruct((NC, OWN, CW), jnp.float32),
        mesh=mesh,
        scratch_types=[pltpu.VMEM((CHS,), jnp.int32),
                       pltpu.VMEM((CHS, CW), jnp.float32),
                       pltpu.VMEM((RPT, CW), jnp.float32),
                       pltpu.VMEM_SHARED((TR, CW), jnp.float32)])


def _s3_body(seg2, cnt2, st1, ig, ibt, w1, b1, w2, b2, g, bt, wm, wb,
             hpo, hqo):
    seg = jnp.concatenate([seg2[0, 0:OWN, :], seg2[1, 0:N - OWN, :]], axis=0)
    cnt = jnp.concatenate([cnt2[0, 0:OWN, 0:1], cnt2[1, 0:N - OWN, 0:1]], axis=0)
    mu1 = st1[0:1, :] / E
    var1 = st1[1:2, :] / E - mu1 * mu1
    s1 = ig[...] * lax.rsqrt(var1 + _EPS)
    sh1 = ibt[...] - mu1 * s1
    node = (seg * s1 + cnt * sh1) / jnp.maximum(cnt, 1.0)
    h = _elu(jnp.dot(node, w1[...], preferred_element_type=jnp.float32) + b1[...])
    h = _elu(jnp.dot(h, w2[...], preferred_element_type=jnp.float32) + b2[...])
    mu = jnp.mean(h, axis=0, keepdims=True)
    var = jnp.mean((h - mu) ** 2, axis=0, keepdims=True)
    h = (h - mu) * lax.rsqrt(var + _EPS) * g[...] + bt[...]
    hpo[...] = jnp.dot(h, wm[...], preferred_element_type=jnp.float32)
    hqo[...] = jnp.dot(h, wb[...], preferred_element_type=jnp.float32)


_stage3 = pl.pallas_call(
    _s3_body,
    grid=(1,),
    in_specs=[pl.BlockSpec((NC, OWN, D), lambda i: (0, 0, 0)),
              pl.BlockSpec((NC, OWN, CW), lambda i: (0, 0, 0)),
              pl.BlockSpec((8, D), lambda i: (0, 0)),
              pl.BlockSpec((1, D), lambda i: (0, 0)),
              pl.BlockSpec((1, D), lambda i: (0, 0)),
              pl.BlockSpec((D, D), lambda i: (0, 0)),
              pl.BlockSpec((1, D), lambda i: (0, 0)),
              pl.BlockSpec((D, D), lambda i: (0, 0)),
              pl.BlockSpec((1, D), lambda i: (0, 0)),
              pl.BlockSpec((1, D), lambda i: (0, 0)),
              pl.BlockSpec((1, D), lambda i: (0, 0)),
              pl.BlockSpec((D, D), lambda i: (0, 0)),
              pl.BlockSpec((D, D), lambda i: (0, 0))],
    out_specs=[pl.BlockSpec((N, D), lambda i: (0, 0)),
               pl.BlockSpec((N, D), lambda i: (0, 0))],
    out_shape=[jax.ShapeDtypeStruct((N, D), jnp.float32),
               jax.ShapeDtypeStruct((N, D), jnp.float32)],
)


def _gather2_body(hp_hbm, hq_hbm, src_hbm, dst_hbm, hps_out, hqd_out,
                  idx1_v, rows1_v, idx2_v, rows2_v, sem1, sem2):
    cid = lax.axis_index("c")
    sid = lax.axis_index("s")
    base = (sid * NC + cid) * EPW

    def body(i, carry):
        off = base + i * CH
        pltpu.sync_copy(src_hbm.at[pl.ds(off, CH)], idx1_v)
        pltpu.sync_copy(dst_hbm.at[pl.ds(off, CH)], idx2_v)
        cp1 = pltpu.async_copy(hp_hbm.at[idx1_v], rows1_v, sem1)
        cp2 = pltpu.async_copy(hq_hbm.at[idx2_v], rows2_v, sem2)
        cp1.wait()
        cp2.wait()
        pltpu.sync_copy(rows1_v, hps_out.at[pl.ds(off, CH), :])
        pltpu.sync_copy(rows2_v, hqd_out.at[pl.ds(off, CH), :])
        return carry

    lax.fori_loop(0, NCH, body, 0)


@functools.cache
def _mk_gather2():
    mesh = plsc.VectorSubcoreMesh(core_axis_name="c", subcore_axis_name="s",
                                  num_cores=NC, num_subcores=NS)
    return pl.kernel(
        _gather2_body,
        out_type=[jax.ShapeDtypeStruct((E, D), jnp.float32),
                  jax.ShapeDtypeStruct((E, D), jnp.float32)],
        mesh=mesh,
        scratch_types=[pltpu.VMEM((CH,), jnp.int32),
                       pltpu.VMEM((CH, D), jnp.float32),
                       pltpu.VMEM((CH,), jnp.int32),
                       pltpu.VMEM((CH, D), jnp.float32),
                       pltpu.SemaphoreType.DMA,
                       pltpu.SemaphoreType.DMA])


def _s5_body(ea, hps, hqd, w1, b1, w2, b2, x2o, st):
    i = pl.program_id(0)
    x1 = _elu(jnp.dot(ea[...], w1[...], preferred_element_type=jnp.float32)
              + hps[...] + hqd[...] + b1[...])
    x2 = _elu(jnp.dot(x1, w2[...], preferred_element_type=jnp.float32) + b2[...])
    x2o[...] = x2

    @pl.when(i == 0)
    def _():
        st[...] = jnp.zeros_like(st)

    st[0:1, :] += jnp.sum(x2, axis=0, keepdims=True)
    st[1:2, :] += jnp.sum(x2 * x2, axis=0, keepdims=True)


_stage5 = pl.pallas_call(
    _s5_body,
    grid=(E // BT,),
    in_specs=[pl.BlockSpec((BT, D), lambda i: (i, 0)),
              pl.BlockSpec((BT, D), lambda i: (i, 0)),
              pl.BlockSpec((BT, D), lambda i: (i, 0)),
              pl.BlockSpec((D, D), lambda i: (0, 0)),
              pl.BlockSpec((1, D), lambda i: (0, 0)),
              pl.BlockSpec((D, D), lambda i: (0, 0)),
              pl.BlockSpec((1, D), lambda i: (0, 0))],
    out_specs=[pl.BlockSpec((BT, D), lambda i: (i, 0)),
               pl.BlockSpec((8, D), lambda i: (0, 0))],
    out_shape=[jax.ShapeDtypeStruct((E, D), jnp.float32),
               jax.ShapeDtypeStruct((8, D), jnp.float32)],
)


def _s6_body(x2, st, g, bt, out):
    mu = st[0:1, :] / E
    var = st[1:2, :] / E - mu * mu
    s = g[...] * lax.rsqrt(var + _EPS)
    out[...] = x2[...] * s + (bt[...] - mu * s)


_stage6 = pl.pallas_call(
    _s6_body,
    grid=(E // BT,),
    in_specs=[pl.BlockSpec((BT, D), lambda i: (i, 0)),
              pl.BlockSpec((8, D), lambda i: (0, 0)),
              pl.BlockSpec((1, D), lambda i: (0, 0)),
              pl.BlockSpec((1, D), lambda i: (0, 0))],
    out_specs=pl.BlockSpec((BT, D), lambda i: (i, 0)),
    out_shape=jax.ShapeDtypeStruct((E, D), jnp.float32),
)


def kernel(edge_attr, edge_index,
           init_W1, init_b1, init_W2, init_b2, init_g, init_bt,
           node_W1, node_b1, node_W2, node_b2, node_g, node_bt,
           edge_W1, edge_b1, edge_W2, edge_b2, edge_g, edge_bt):
    src = edge_index[0]
    dst = edge_index[1]
    r1 = lambda v: v.reshape(1, D)

    h2a, st1 = _stage1(edge_attr, init_W1, r1(init_b1), init_W2, r1(init_b2))

    zrow = jnp.zeros((RPT, D), jnp.float32)
    zcnt = jnp.zeros((RPT, CW), jnp.float32)
    ones = jnp.ones((CHS, CW), jnp.float32)
    seg = _mk_seg_scatter()(h2a, dst, zrow)
    cnt = _mk_cnt_scatter()(dst, zcnt, ones)

    w1t = edge_W1[0:D]
    w1m = edge_W1[D:2 * D]
    w1b = edge_W1[2 * D:3 * D]
    hp, hq = _stage3(seg, cnt, st1, r1(init_g), r1(init_bt),
                     node_W1, r1(node_b1), node_W2, r1(node_b2),
                     r1(node_g), r1(node_bt), w1m, w1b)

    hps, hqd = _mk_gather2()(hp, hq, src, dst)

    x2, st5 = _stage5(edge_attr, hps, hqd, w1t, r1(edge_b1),
                      edge_W2, r1(edge_b2))
    return _stage6(x2, st5, r1(edge_g), r1(edge_bt))
